# baseline (speedup 1.0000x reference)
import functools

import jax
import jax.numpy as jnp
from jax import lax
from jax.experimental import pallas as pl
from jax.experimental.pallas import tpu as pltpu
from jax.experimental.pallas import tpu_sc as plsc

VOCAB = 100000
EMBED = 128
SEQ = 2048
BATCH = 4
N_SEG = 3
ROWS = SEQ * BATCH
NC, NS = 2, 16
NW = NC * NS
SPW = SEQ // NW
RPW = SPW * BATCH
SCHUNK = SPW // 2
NCHUNK = 2
RCHUNK = BATCH * SCHUNK
LANES = 16
NG = EMBED // LANES
SPB = LANES // BATCH
SCALE = float(EMBED) ** 0.5
EPS = 1e-5

GRID = 8
TROWS = ROWS // GRID
TSEQ = SEQ // GRID


@functools.partial(
    pl.kernel,
    out_type=jax.ShapeDtypeStruct((ROWS, EMBED), jnp.float32),
    mesh=plsc.VectorSubcoreMesh(
        core_axis_name="c", subcore_axis_name="s", num_cores=NC, num_subcores=NS
    ),
    compiler_params=pltpu.CompilerParams(needs_layout_passes=False),
    scratch_types=[
        pltpu.VMEM((SPW, BATCH), jnp.int32),
        pltpu.VMEM((SPW, BATCH), jnp.int32),
        pltpu.VMEM((NCHUNK, RCHUNK), jnp.int32),
        pltpu.VMEM((NCHUNK, RCHUNK), jnp.int32),
        pltpu.VMEM((RPW, EMBED), jnp.float32),
        pltpu.VMEM((RPW, EMBED), jnp.float32),
        pltpu.SemaphoreType.DMA,
        pltpu.SemaphoreType.DMA,
        pltpu.SemaphoreType.DMA,
    ],
)
def _gather_kernel(all_idx, tok_tab, pos_tab, out, idx_t, idx_p, idxl_t,
                   idxl_p, rows_t, rows_p, sem_g0, sem_g1, sem_w):
    wid = lax.axis_index("s") * NC + lax.axis_index("c")
    base = wid * SPW

    pltpu.sync_copy(all_idx.at[0, pl.ds(base, SPW)], idx_t)
    pltpu.sync_copy(all_idx.at[1, pl.ds(base, SPW)], idx_p)

    iota = lax.iota(jnp.int32, LANES)
    lane_sp = iota // BATCH
    lane_b = iota % BATCH

    for k in range(RPW // LANES):
        rows = SPB * k + lane_sp
        vt = plsc.load_gather(idx_t, [rows, lane_b])
        vp = plsc.load_gather(idx_p, [rows, lane_b])
        j, off = divmod(k * LANES, RCHUNK)
        idxl_t[j, pl.ds(off, LANES)] = vt
        idxl_p[j, pl.ds(off, LANES)] = vp

    gsems = [sem_g0, sem_g1]
    gcopies = []
    for j in range(NCHUNK):
        sl = pl.ds(j * RCHUNK, RCHUNK)
        gcopies.append((
            pltpu.async_copy(tok_tab.at[idxl_t.at[j]], rows_t.at[sl], gsems[j]),
            pltpu.async_copy(pos_tab.at[idxl_p.at[j]], rows_p.at[sl], gsems[j]),
        ))

    wb = []
    for j in range(NCHUNK):
        for c in gcopies[j]:
            c.wait()

        @plsc.parallel_loop(j * RCHUNK, (j + 1) * RCHUNK, step=1, unroll=8)
        def row_body(r):
            for g in range(NG):
                sl = pl.ds(g * LANES, LANES)
                rows_t[r, sl] = rows_t[r, sl] * SCALE + rows_p[r, sl]

        sl = pl.ds(j * RCHUNK, RCHUNK)
        wb.append(pltpu.async_copy(
            rows_t.at[sl], out.at[pl.ds(wid * RPW + j * RCHUNK, RCHUNK)], sem_w))
    for c in wb:
        c.wait()


def _ln_body(sum_ref, seg_idx_ref, seg_tab_ref, gamma_ref, beta_ref, out_ref):
    x3 = sum_ref[...].reshape(TSEQ, BATCH, EMBED)
    si = seg_idx_ref[...][:, :, None]
    seg = seg_tab_ref[...]
    s0 = seg[0][None, None, :]
    s1 = seg[1][None, None, :]
    s2 = seg[2][None, None, :]
    x3 = x3 + jnp.where(si == 0, s0, jnp.where(si == 1, s1, s2))
    x = x3.reshape(TROWS, EMBED)
    ones = jnp.full((EMBED, EMBED), 1.0 / EMBED, jnp.float32)
    mean = jax.lax.dot(x, ones, precision=jax.lax.Precision.HIGHEST)
    sq = jax.lax.dot(x * x, ones, precision=jax.lax.Precision.HIGHEST)
    var = sq - mean * mean
    y = (x - mean) * lax.rsqrt(var + EPS)
    y = y * gamma_ref[...][None, :] + beta_ref[...][None, :]
    out_ref[...] = y.reshape(TSEQ, BATCH, EMBED)


_ln_kernel = pl.pallas_call(
    _ln_body,
    grid=(GRID,),
    in_specs=[
        pl.BlockSpec((TROWS, EMBED), lambda i: (i, 0)),
        pl.BlockSpec((TSEQ, BATCH), lambda i: (i, 0)),
        pl.BlockSpec((N_SEG, EMBED), lambda i: (0, 0)),
        pl.BlockSpec((EMBED,), lambda i: (0,)),
        pl.BlockSpec((EMBED,), lambda i: (0,)),
    ],
    out_specs=pl.BlockSpec((TSEQ, BATCH, EMBED), lambda i: (i, 0, 0)),
    out_shape=jax.ShapeDtypeStruct((SEQ, BATCH, EMBED), jnp.float32),
)


def kernel(token_sequence, segment_indices, position_indices, token_table,
           segment_table, position_table, ln_gamma, ln_beta):
    all_idx = jnp.stack([token_sequence.astype(jnp.int32),
                         position_indices.astype(jnp.int32)])
    summed = _gather_kernel(all_idx, token_table, position_table)
    return _ln_kernel(summed, segment_indices.astype(jnp.int32),
                      segment_table, ln_gamma, ln_beta)


# speedup vs baseline: 1.1120x; 1.1120x over previous
import functools

import jax
import jax.numpy as jnp
from jax import lax
from jax.experimental import pallas as pl
from jax.experimental.pallas import tpu as pltpu
from jax.experimental.pallas import tpu_sc as plsc

VOCAB = 100000
EMBED = 128
SEQ = 2048
BATCH = 4
N_SEG = 3
ROWS = SEQ * BATCH
NC, NS = 2, 16
NW = NC * NS
SPW = SEQ // NW
RPW = SPW * BATCH
SCHUNK = SPW // 2
NCHUNK = 2
RCHUNK = BATCH * SCHUNK
LANES = 16
NG = EMBED // LANES
SPB = LANES // BATCH
SCALE = float(EMBED) ** 0.5
EPS = 1e-5

GRID = 8
TROWS = ROWS // GRID
TSEQ = SEQ // GRID


@functools.partial(
    pl.kernel,
    out_type=jax.ShapeDtypeStruct((ROWS, EMBED), jnp.float32),
    mesh=plsc.VectorSubcoreMesh(
        core_axis_name="c", subcore_axis_name="s", num_cores=NC, num_subcores=NS
    ),
    compiler_params=pltpu.CompilerParams(needs_layout_passes=False),
    scratch_types=[
        pltpu.VMEM((SPW, BATCH), jnp.int32),
        pltpu.VMEM((SPW, BATCH), jnp.int32),
        pltpu.VMEM((NCHUNK, RCHUNK), jnp.int32),
        pltpu.VMEM((NCHUNK, RCHUNK), jnp.int32),
        pltpu.VMEM((RPW, EMBED), jnp.float32),
        pltpu.VMEM((RPW, EMBED), jnp.float32),
        pltpu.SemaphoreType.DMA,
        pltpu.SemaphoreType.DMA,
        pltpu.SemaphoreType.DMA,
    ],
)
def _gather_kernel(all_idx, tok_tab, pos_tab, out, idx_t, idx_p, idxl_t,
                   idxl_p, rows_t, rows_p, sem_g0, sem_g1, sem_w):
    wid = lax.axis_index("s") * NC + lax.axis_index("c")
    base = wid * SPW

    pltpu.sync_copy(all_idx.at[0, pl.ds(base, SPW)], idx_t)
    pltpu.sync_copy(all_idx.at[1, pl.ds(base, SPW)], idx_p)

    iota = lax.iota(jnp.int32, LANES)
    lane_sp = iota // BATCH
    lane_b = iota % BATCH

    for k in range(RPW // LANES):
        rows = SPB * k + lane_sp
        vt = plsc.load_gather(idx_t, [rows, lane_b])
        vp = plsc.load_gather(idx_p, [rows, lane_b])
        j, off = divmod(k * LANES, RCHUNK)
        idxl_t[j, pl.ds(off, LANES)] = vt
        idxl_p[j, pl.ds(off, LANES)] = vp

    gsems = [sem_g0, sem_g1]
    gcopies = []
    for j in range(NCHUNK):
        sl = pl.ds(j * RCHUNK, RCHUNK)
        gcopies.append((
            pltpu.async_copy(tok_tab.at[idxl_t.at[j]], rows_t.at[sl], gsems[j]),
            pltpu.async_copy(pos_tab.at[idxl_p.at[j]], rows_p.at[sl], gsems[j]),
        ))

    wb = []
    for j in range(NCHUNK):
        for c in gcopies[j]:
            c.wait()

        @plsc.parallel_loop(j * RCHUNK, (j + 1) * RCHUNK, step=1, unroll=8)
        def row_body(r):
            for g in range(NG):
                sl = pl.ds(g * LANES, LANES)
                rows_t[r, sl] = rows_t[r, sl] * SCALE + rows_p[r, sl]

        sl = pl.ds(j * RCHUNK, RCHUNK)
        wb.append(pltpu.async_copy(
            rows_t.at[sl], out.at[pl.ds(wid * RPW + j * RCHUNK, RCHUNK)], sem_w))
    for c in wb:
        c.wait()


def _ln_body(sum_ref, seg_idx_ref, seg_tab_ref, gamma_ref, beta_ref, out_ref):
    x3 = sum_ref[...].reshape(TSEQ, BATCH, EMBED)
    si = seg_idx_ref[...][:, :, None]
    seg = seg_tab_ref[...]
    s0 = seg[0][None, None, :]
    s1 = seg[1][None, None, :]
    s2 = seg[2][None, None, :]
    x3 = x3 + jnp.where(si == 0, s0, jnp.where(si == 1, s1, s2))
    x = x3.reshape(TROWS, EMBED)
    ones = jnp.full((EMBED, EMBED), 1.0 / EMBED, jnp.float32)
    mean = jax.lax.dot(x, ones)
    sq = jax.lax.dot(x * x, ones)
    var = sq - mean * mean
    y = (x - mean) * lax.rsqrt(var + EPS)
    y = y * gamma_ref[...][None, :] + beta_ref[...][None, :]
    out_ref[...] = y.reshape(TSEQ, BATCH, EMBED)


_ln_kernel = pl.pallas_call(
    _ln_body,
    grid=(GRID,),
    in_specs=[
        pl.BlockSpec((TROWS, EMBED), lambda i: (i, 0)),
        pl.BlockSpec((TSEQ, BATCH), lambda i: (i, 0)),
        pl.BlockSpec((N_SEG, EMBED), lambda i: (0, 0)),
        pl.BlockSpec((EMBED,), lambda i: (0,)),
        pl.BlockSpec((EMBED,), lambda i: (0,)),
    ],
    out_specs=pl.BlockSpec((TSEQ, BATCH, EMBED), lambda i: (i, 0, 0)),
    out_shape=jax.ShapeDtypeStruct((SEQ, BATCH, EMBED), jnp.float32),
)


def kernel(token_sequence, segment_indices, position_indices, token_table,
           segment_table, position_table, ln_gamma, ln_beta):
    all_idx = jnp.stack([token_sequence.astype(jnp.int32),
                         position_indices.astype(jnp.int32)])
    summed = _gather_kernel(all_idx, token_table, position_table)
    return _ln_kernel(summed, segment_indices.astype(jnp.int32),
                      segment_table, ln_gamma, ln_beta)
